# baseline (speedup 1.0000x reference)
import functools

import jax
import jax.numpy as jnp
from jax import lax
from jax.experimental import pallas as pl
from jax.experimental.pallas import tpu as pltpu
from jax.experimental.pallas import tpu_sc as plsc

D = 128
NC = 2
NS = 16
NW = NC * NS
CHUNK = 64
RDEPTH = 4
IDEPTH = 8


def _matmul_body(x_ref, w_ref, o_ref):
    o_ref[...] = jnp.dot(x_ref[...], w_ref[...],
                         preferred_element_type=jnp.float32)


def _finalize_body(acc_ref, cnt_ref, b_ref, o_ref):
    a = acc_ref[0, :, :] + acc_ref[1, :, :]
    cnt = jnp.sum(cnt_ref[...], axis=1, keepdims=True)
    scale = lax.rsqrt(cnt + 1.0)
    o_ref[...] = jnp.maximum(a * scale + b_ref[...], 0.0)


def _make_sc_scatter(n_chunks, n_acc, n_deg):
    mesh = plsc.VectorSubcoreMesh(core_axis_name="c", subcore_axis_name="s")
    per_tile = n_acc // NS

    @functools.partial(
        pl.kernel,
        mesh=mesh,
        compiler_params=pltpu.CompilerParams(needs_layout_passes=False),
        out_type=[
            jax.ShapeDtypeStruct((NC, n_acc, D), jnp.float32),
            jax.ShapeDtypeStruct((NW, n_deg), jnp.float32),
        ],
        scratch_types=[
            pltpu.VMEM((IDEPTH, CHUNK), jnp.int32),
            pltpu.VMEM((IDEPTH, CHUNK), jnp.int32),
        ] + [
            pltpu.VMEM((CHUNK, D), jnp.float32)
            for _ in range(RDEPTH)
        ] + [
            pltpu.VMEM((n_deg,), jnp.float32),
            pltpu.VMEM_SHARED((n_acc, D), jnp.float32),
            pltpu.SemaphoreType.DMA,
            pltpu.SemaphoreType.DMA,
            pltpu.SemaphoreType.DMA,
        ],
    )
    def sc_scatter(src_hbm, dst_hbm, h_hbm, acc_hbm, cnt_hbm,
                   sidx, didx, *rest):
        rows = rest[:RDEPTH]
        deg_v, acc_sh, gsem, isems, isemd = rest[RDEPTH:]
        cid = lax.axis_index("c")
        sid = lax.axis_index("s")
        wid = cid * NS + sid
        rows0 = rows[0]

        def _zero_row(i, carry):
            for v in range(D // 16):
                rows0[i, pl.ds(v * 16, 16)] = jnp.zeros((16,), jnp.float32)
            return carry
        lax.fori_loop(0, CHUNK, _zero_row, 0)

        def _zero_deg(i, carry):
            deg_v[pl.ds(i * 16, 16)] = jnp.zeros((16,), jnp.float32)
            return carry
        lax.fori_loop(0, n_deg // 16, _zero_deg, 0)

        zbase = sid * per_tile
        for k in range(per_tile // CHUNK):
            pltpu.sync_copy(rows0, acc_sh.at[pl.ds(zbase + k * CHUNK, CHUNK)])
        plsc.subcore_barrier()

        for c in range(IDEPTH):
            pltpu.async_copy(src_hbm.at[wid, c], sidx.at[c], isems)
            pltpu.async_copy(dst_hbm.at[wid, c], didx.at[c], isemd)
        for c in range(RDEPTH):
            pltpu.make_async_copy(src_hbm.at[wid, 0], sidx.at[c], isems).wait()
            pltpu.make_async_copy(dst_hbm.at[wid, 0], didx.at[c], isemd).wait()
            pltpu.async_copy(h_hbm.at[sidx.at[c]], rows[c], gsem)

        ones16 = jnp.ones((16,), jnp.float32)
        rows_dummy = h_hbm.at[pl.ds(0, CHUNK)]

        def _step(i, carry):
            for u in range(IDEPTH):
                j = i * IDEPTH + u
                buf = rows[u % RDEPTH]
                pltpu.make_async_copy(rows_dummy, buf, gsem).wait()
                pltpu.sync_copy(buf, acc_sh.at[didx.at[u]], add=True)
                for v in range(CHUNK // 16):
                    dv = didx[u, pl.ds(v * 16, 16)]
                    plsc.addupdate_scatter(deg_v, [dv], ones16)

                @pl.when(j + IDEPTH < n_chunks)
                def _():
                    pltpu.async_copy(
                        src_hbm.at[wid, j + IDEPTH], sidx.at[u], isems)
                    pltpu.async_copy(
                        dst_hbm.at[wid, j + IDEPTH], didx.at[u], isemd)

                @pl.when(j + RDEPTH < n_chunks)
                def _():
                    pltpu.make_async_copy(
                        src_hbm.at[wid, 0], sidx.at[u], isems).wait()
                    pltpu.make_async_copy(
                        dst_hbm.at[wid, 0], didx.at[u], isemd).wait()
                    pltpu.async_copy(
                        h_hbm.at[sidx.at[(u + RDEPTH) % IDEPTH]], buf, gsem)
            return carry
        lax.fori_loop(0, n_chunks // IDEPTH, _step, 0)

        pltpu.sync_copy(deg_v, cnt_hbm.at[wid])
        plsc.subcore_barrier()

        pltpu.sync_copy(acc_sh.at[pl.ds(zbase, per_tile)],
                        acc_hbm.at[cid, pl.ds(zbase, per_tile)])

    return sc_scatter


def kernel(x, edge_index, W, b):
    n_src, d_in = x.shape
    n_out = n_src
    e = edge_index.shape[1]

    per_w = -(-e // NW)
    n_chunks = -(-(-(-per_w // CHUNK)) // IDEPTH) * IDEPTH
    per_w = n_chunks * CHUNK
    e_pad = per_w * NW

    trash = n_out
    n_acc = -(-(n_out + 1) // (NS * CHUNK)) * (NS * CHUNK)
    src = edge_index[0]
    dst = edge_index[1]
    pad = e_pad - e
    src_p = jnp.concatenate([src, jnp.zeros((pad,), jnp.int32)])
    dst_p = jnp.concatenate([dst, jnp.full((pad,), trash, jnp.int32)])
    src3 = src_p.reshape(NW, n_chunks, CHUNK)
    dst3 = dst_p.reshape(NW, n_chunks, CHUNK)

    blk = 2000
    h = pl.pallas_call(
        _matmul_body,
        grid=(n_src // blk,),
        in_specs=[
            pl.BlockSpec((blk, d_in), lambda i: (i, 0)),
            pl.BlockSpec((d_in, D), lambda i: (0, 0)),
        ],
        out_specs=pl.BlockSpec((blk, D), lambda i: (i, 0)),
        out_shape=jax.ShapeDtypeStruct((n_src, D), jnp.float32),
    )(x, W)

    n_deg = -(-(n_out + 1) // 16) * 16
    acc, cnt = _make_sc_scatter(n_chunks, n_acc, n_deg)(src3, dst3, h)
    cnt_t = cnt.T

    fblk = 2000
    out = pl.pallas_call(
        _finalize_body,
        grid=(n_out // fblk,),
        in_specs=[
            pl.BlockSpec((NC, fblk, D), lambda i: (0, i, 0)),
            pl.BlockSpec((fblk, NW), lambda i: (i, 0)),
            pl.BlockSpec((1, D), lambda i: (0, 0)),
        ],
        out_specs=pl.BlockSpec((fblk, D), lambda i: (i, 0)),
        out_shape=jax.ShapeDtypeStruct((n_out, D), jnp.float32),
    )(acc, cnt_t, b.reshape(1, D))
    return out


# speedup vs baseline: 1.1049x; 1.1049x over previous
import functools

import jax
import jax.numpy as jnp
from jax import lax
from jax.experimental import pallas as pl
from jax.experimental.pallas import tpu as pltpu
from jax.experimental.pallas import tpu_sc as plsc

D = 128
NC = 2
NS = 16
NW = NC * NS
CHUNK = 80
RDEPTH = 2
IDEPTH = 8
HROWS = 3456
NPASS = 3
DEGCH = 1024


def _matmul_body(x_ref, w_ref, o_ref):
    o_ref[...] = jnp.dot(x_ref[...], w_ref[...],
                         preferred_element_type=jnp.float32)


def _finalize_body(acc_ref, cnt_ref, b_ref, o_ref):
    a = acc_ref[0, :, :] + acc_ref[1, :, :]
    cnt = jnp.sum(cnt_ref[...], axis=1, keepdims=True)
    scale = lax.rsqrt(cnt + 1.0)
    o_ref[...] = jnp.maximum(a * scale + b_ref[...], 0.0)


def _make_deg(n_flat, n_deg):
    mesh = plsc.VectorSubcoreMesh(core_axis_name="c", subcore_axis_name="s")

    @functools.partial(
        pl.kernel,
        mesh=mesh,
        compiler_params=pltpu.CompilerParams(needs_layout_passes=False),
        out_type=jax.ShapeDtypeStruct((NW, n_deg), jnp.float32),
        scratch_types=[
            pltpu.VMEM((DEGCH,), jnp.int32),
            pltpu.VMEM((DEGCH,), jnp.int32),
            pltpu.VMEM((n_deg,), jnp.float32),
            pltpu.SemaphoreType.DMA,
        ],
    )
    def deg_kernel(dst_hbm, cnt_hbm, dbuf0, dbuf1, deg_v, dsem):
        cid = lax.axis_index("c")
        sid = lax.axis_index("s")
        wid = cid * NS + sid

        def _zero_deg(i, carry):
            deg_v[pl.ds(i * 16, 16)] = jnp.zeros((16,), jnp.float32)
            return carry
        lax.fori_loop(0, n_deg // 16, _zero_deg, 0)

        pltpu.async_copy(dst_hbm.at[wid, pl.ds(0, DEGCH)], dbuf0, dsem)
        pltpu.async_copy(dst_hbm.at[wid, pl.ds(DEGCH, DEGCH)], dbuf1, dsem)
        bufs = (dbuf0, dbuf1)
        dummy = dst_hbm.at[0, pl.ds(0, DEGCH)]
        ones16 = jnp.ones((16,), jnp.float32)

        def _pair(i, carry):
            for u in range(2):
                k = i * 2 + u
                buf = bufs[u]
                pltpu.make_async_copy(dummy, buf, dsem).wait()

                def _hist(g, carry2):
                    dv = buf[pl.ds(g * 16, 16)]
                    plsc.addupdate_scatter(deg_v, [dv], ones16)
                    return carry2
                lax.fori_loop(0, DEGCH // 16, _hist, 0)

                @pl.when(k + 2 < n_flat)
                def _():
                    pltpu.async_copy(
                        dst_hbm.at[wid, pl.ds((k + 2) * DEGCH, DEGCH)],
                        buf, dsem)
            return carry
        lax.fori_loop(0, n_flat // 2, _pair, 0)
        pltpu.sync_copy(deg_v, cnt_hbm.at[wid])

    return deg_kernel


def _make_sc_scatter(n_chunks, n_acc):
    mesh = plsc.VectorSubcoreMesh(core_axis_name="c", subcore_axis_name="s")
    per_tile = n_acc // NS
    hrows_pt = HROWS // NS
    trash = n_acc - 64

    @functools.partial(
        pl.kernel,
        mesh=mesh,
        compiler_params=pltpu.CompilerParams(needs_layout_passes=False),
        out_type=jax.ShapeDtypeStruct((NC, n_acc, D), jnp.float32),
        scratch_types=[
            pltpu.VMEM((IDEPTH, CHUNK), jnp.int32),
            pltpu.VMEM((IDEPTH, CHUNK), jnp.int32),
            pltpu.VMEM((CHUNK, D), jnp.float32),
            pltpu.VMEM((CHUNK, D), jnp.float32),
            pltpu.VMEM_SHARED((HROWS, D), jnp.float32),
            pltpu.VMEM_SHARED((n_acc, D), jnp.float32),
            pltpu.SemaphoreType.DMA,
            pltpu.SemaphoreType.DMA,
            pltpu.SemaphoreType.DMA,
        ],
    )
    def sc_scatter(src_hbm, dst_hbm, h_hbm, acc_hbm,
                   sidx, didx, rows0, rows1, h_sh, acc_sh,
                   gsem, isems, isemd):
        cid = lax.axis_index("c")
        sid = lax.axis_index("s")
        wid = cid * NS + sid
        rows = (rows0, rows1)

        def _transform(s, q):
            for v in range(CHUNK // 16):
                sl = pl.ds(v * 16, 16)
                sv = sidx[s, sl]
                dv = didx[s, sl]
                ls = sv - (q * HROWS)
                ok = (ls >= 0) & (ls < HROWS)
                sidx[s, sl] = jnp.where(ok, ls, 0)
                didx[s, sl] = jnp.where(ok, dv, trash + (dv & 63))

        def _zero_row(i, carry):
            for v in range(D // 16):
                rows0[i, pl.ds(v * 16, 16)] = jnp.zeros((16,), jnp.float32)
            return carry
        lax.fori_loop(0, CHUNK, _zero_row, 0)

        zbase = sid * per_tile
        nfull, tail = divmod(per_tile, CHUNK)
        for k in range(nfull):
            pltpu.sync_copy(rows0, acc_sh.at[pl.ds(zbase + k * CHUNK, CHUNK)])
        if tail:
            pltpu.sync_copy(rows0.at[pl.ds(0, tail)],
                            acc_sh.at[pl.ds(zbase + nfull * CHUNK, tail)])

        rows_dummy = h_hbm.at[pl.ds(0, CHUNK)]
        sidx_dummy = src_hbm.at[0, 0]
        didx_dummy = dst_hbm.at[0, 0]

        for q in range(NPASS):
            plsc.subcore_barrier()
            pltpu.sync_copy(
                h_hbm.at[pl.ds(q * HROWS + sid * hrows_pt, hrows_pt)],
                h_sh.at[pl.ds(sid * hrows_pt, hrows_pt)])
            plsc.subcore_barrier()

            for c in range(IDEPTH):
                pltpu.async_copy(src_hbm.at[wid, c], sidx.at[c], isems)
                pltpu.async_copy(dst_hbm.at[wid, c], didx.at[c], isemd)
            for c in range(RDEPTH):
                pltpu.make_async_copy(sidx_dummy, sidx.at[c], isems).wait()
                pltpu.make_async_copy(didx_dummy, didx.at[c], isemd).wait()
                _transform(c, q)
                pltpu.async_copy(h_sh.at[sidx.at[c]], rows[c], gsem)

            def _step(i, carry):
                for u in range(IDEPTH):
                    j = i * IDEPTH + u
                    buf = rows[u % RDEPTH]
                    pltpu.make_async_copy(rows_dummy, buf, gsem).wait()
                    pltpu.sync_copy(buf, acc_sh.at[didx.at[u]], add=True)

                    @pl.when(j + IDEPTH < n_chunks)
                    def _():
                        pltpu.async_copy(
                            src_hbm.at[wid, j + IDEPTH], sidx.at[u], isems)
                        pltpu.async_copy(
                            dst_hbm.at[wid, j + IDEPTH], didx.at[u], isemd)

                    @pl.when(j + RDEPTH < n_chunks)
                    def _():
                        pltpu.make_async_copy(
                            sidx_dummy, sidx.at[u], isems).wait()
                        pltpu.make_async_copy(
                            didx_dummy, didx.at[u], isemd).wait()
                        _transform((u + RDEPTH) % IDEPTH, q)
                        pltpu.async_copy(
                            h_sh.at[sidx.at[(u + RDEPTH) % IDEPTH]],
                            buf, gsem)
                return carry
            lax.fori_loop(0, n_chunks // IDEPTH, _step, 0)

        plsc.subcore_barrier()
        pltpu.sync_copy(acc_sh.at[pl.ds(zbase, per_tile)],
                        acc_hbm.at[cid, pl.ds(zbase, per_tile)])

    return sc_scatter


def kernel(x, edge_index, W, b):
    n_src, d_in = x.shape
    n_out = n_src
    e = edge_index.shape[1]

    per_w = -(-e // NW)
    n_chunks = -(-(-(-per_w // CHUNK)) // IDEPTH) * IDEPTH
    per_w = n_chunks * CHUNK
    e_pad = per_w * NW

    n_acc = -(-(n_out + 1 + 64) // 128) * 128
    src = edge_index[0]
    dst = edge_index[1]
    pad = e_pad - e
    src_p = jnp.concatenate([src, jnp.zeros((pad,), jnp.int32)])
    dst_p = jnp.concatenate([dst, jnp.full((pad,), n_out, jnp.int32)])
    src3 = src_p.reshape(NW, n_chunks, CHUNK)
    dst3 = dst_p.reshape(NW, n_chunks, CHUNK)
    dst2 = dst_p.reshape(NW, per_w)

    blk = 2000
    h = pl.pallas_call(
        _matmul_body,
        grid=(n_src // blk,),
        in_specs=[
            pl.BlockSpec((blk, d_in), lambda i: (i, 0)),
            pl.BlockSpec((d_in, D), lambda i: (0, 0)),
        ],
        out_specs=pl.BlockSpec((blk, D), lambda i: (i, 0)),
        out_shape=jax.ShapeDtypeStruct((n_src, D), jnp.float32),
    )(x, W)
    h_pad = jnp.concatenate(
        [h, jnp.zeros((NPASS * HROWS - n_src, D), jnp.float32)])

    n_deg = -(-(n_out + 1) // 16) * 16
    cnt = _make_deg(per_w // DEGCH, n_deg)(dst2)
    acc = _make_sc_scatter(n_chunks, n_acc)(src3, dst3, h_pad)
    cnt_t = cnt.T

    fblk = 2000
    out = pl.pallas_call(
        _finalize_body,
        grid=(n_out // fblk,),
        in_specs=[
            pl.BlockSpec((NC, fblk, D), lambda i: (0, i, 0)),
            pl.BlockSpec((fblk, NW), lambda i: (i, 0)),
            pl.BlockSpec((1, D), lambda i: (0, 0)),
        ],
        out_specs=pl.BlockSpec((fblk, D), lambda i: (i, 0)),
        out_shape=jax.ShapeDtypeStruct((n_out, D), jnp.float32),
    )(acc, cnt_t, b.reshape(1, D))
    return out


# speedup vs baseline: 1.1169x; 1.0109x over previous
import functools

import jax
import jax.numpy as jnp
from jax import lax
from jax.experimental import pallas as pl
from jax.experimental.pallas import tpu as pltpu
from jax.experimental.pallas import tpu_sc as plsc

D = 128
NC = 2
NS = 16
NW = NC * NS
CHUNK = 80
RDEPTH = 2
IDEPTH = 8
HROWS = 3456
NPASS = 3
DEGCH = 1024


def _matmul_body(x_ref, w_ref, o_ref):
    o_ref[...] = jnp.dot(x_ref[...], w_ref[...],
                         preferred_element_type=jnp.float32)


def _finalize_body(acc_ref, cnt_ref, b_ref, o_ref):
    a = acc_ref[0, :, :] + acc_ref[1, :, :]
    cnt = jnp.sum(cnt_ref[...], axis=1, keepdims=True)
    scale = lax.rsqrt(cnt + 1.0)
    o_ref[...] = jnp.maximum(a * scale + b_ref[...], 0.0)


def _make_deg(n_flat, n_deg):
    mesh = plsc.VectorSubcoreMesh(core_axis_name="c", subcore_axis_name="s")

    @functools.partial(
        pl.kernel,
        mesh=mesh,
        compiler_params=pltpu.CompilerParams(needs_layout_passes=False),
        out_type=jax.ShapeDtypeStruct((NW, n_deg), jnp.float32),
        scratch_types=[
            pltpu.VMEM((DEGCH,), jnp.int32),
            pltpu.VMEM((DEGCH,), jnp.int32),
            pltpu.VMEM((n_deg,), jnp.float32),
            pltpu.SemaphoreType.DMA,
        ],
    )
    def deg_kernel(dst_hbm, cnt_hbm, dbuf0, dbuf1, deg_v, dsem):
        cid = lax.axis_index("c")
        sid = lax.axis_index("s")
        wid = cid * NS + sid

        def _zero_deg(i, carry):
            deg_v[pl.ds(i * 16, 16)] = jnp.zeros((16,), jnp.float32)
            return carry
        lax.fori_loop(0, n_deg // 16, _zero_deg, 0)

        pltpu.async_copy(dst_hbm.at[wid, pl.ds(0, DEGCH)], dbuf0, dsem)
        pltpu.async_copy(dst_hbm.at[wid, pl.ds(DEGCH, DEGCH)], dbuf1, dsem)
        bufs = (dbuf0, dbuf1)
        dummy = dst_hbm.at[0, pl.ds(0, DEGCH)]
        ones16 = jnp.ones((16,), jnp.float32)

        def _pair(i, carry):
            for u in range(2):
                k = i * 2 + u
                buf = bufs[u]
                pltpu.make_async_copy(dummy, buf, dsem).wait()

                def _hist(g, carry2):
                    dv = buf[pl.ds(g * 16, 16)]
                    plsc.addupdate_scatter(deg_v, [dv], ones16)
                    return carry2
                lax.fori_loop(0, DEGCH // 16, _hist, 0)

                @pl.when(k + 2 < n_flat)
                def _():
                    pltpu.async_copy(
                        dst_hbm.at[wid, pl.ds((k + 2) * DEGCH, DEGCH)],
                        buf, dsem)
            return carry
        lax.fori_loop(0, n_flat // 2, _pair, 0)
        pltpu.sync_copy(deg_v, cnt_hbm.at[wid])

    return deg_kernel


def _make_sc_scatter(n_chunks, n_acc):
    mesh = plsc.VectorSubcoreMesh(core_axis_name="c", subcore_axis_name="s")
    per_tile = n_acc // NS
    hrows_pt = HROWS // NS
    trash = n_acc - 64

    @functools.partial(
        pl.kernel,
        mesh=mesh,
        compiler_params=pltpu.CompilerParams(needs_layout_passes=False),
        out_type=jax.ShapeDtypeStruct((NC, n_acc, D), jnp.float32),
        scratch_types=[
            pltpu.VMEM((IDEPTH, CHUNK), jnp.int32),
            pltpu.VMEM((IDEPTH, CHUNK), jnp.int32),
            pltpu.VMEM((CHUNK, D), jnp.float32),
            pltpu.VMEM((CHUNK, D), jnp.float32),
            pltpu.VMEM_SHARED((HROWS, D), jnp.float32),
            pltpu.VMEM_SHARED((n_acc, D), jnp.float32),
            pltpu.SemaphoreType.DMA,
            pltpu.SemaphoreType.DMA,
            pltpu.SemaphoreType.DMA,
            pltpu.SemaphoreType.DMA,
        ],
    )
    def sc_scatter(src_hbm, dst_hbm, h_hbm, acc_hbm,
                   sidx, didx, rows0, rows1, h_sh, acc_sh,
                   gsem, isems, isemd, ssem):
        cid = lax.axis_index("c")
        sid = lax.axis_index("s")
        wid = cid * NS + sid
        rows = (rows0, rows1)

        def _transform(s, q):
            for v in range(CHUNK // 16):
                sl = pl.ds(v * 16, 16)
                sv = sidx[s, sl]
                dv = didx[s, sl]
                ls = sv - (q * HROWS)
                ok = (ls >= 0) & (ls < HROWS)
                sidx[s, sl] = jnp.where(ok, ls, 0)
                didx[s, sl] = jnp.where(ok, dv, trash + (dv & 63))

        def _zero_row(i, carry):
            for v in range(D // 16):
                rows0[i, pl.ds(v * 16, 16)] = jnp.zeros((16,), jnp.float32)
            return carry
        lax.fori_loop(0, CHUNK, _zero_row, 0)

        zbase = sid * per_tile
        nfull, tail = divmod(per_tile, CHUNK)
        for k in range(nfull):
            pltpu.sync_copy(rows0, acc_sh.at[pl.ds(zbase + k * CHUNK, CHUNK)])
        if tail:
            pltpu.sync_copy(rows0.at[pl.ds(0, tail)],
                            acc_sh.at[pl.ds(zbase + nfull * CHUNK, tail)])

        rows_dummy = h_hbm.at[pl.ds(0, CHUNK)]
        sidx_dummy = src_hbm.at[0, 0]
        didx_dummy = dst_hbm.at[0, 0]

        for q in range(NPASS):
            plsc.subcore_barrier()
            pltpu.sync_copy(
                h_hbm.at[pl.ds(q * HROWS + sid * hrows_pt, hrows_pt)],
                h_sh.at[pl.ds(sid * hrows_pt, hrows_pt)])
            plsc.subcore_barrier()

            for c in range(IDEPTH):
                pltpu.async_copy(src_hbm.at[wid, c], sidx.at[c], isems)
                pltpu.async_copy(dst_hbm.at[wid, c], didx.at[c], isemd)
            for c in range(RDEPTH):
                pltpu.make_async_copy(sidx_dummy, sidx.at[c], isems).wait()
                pltpu.make_async_copy(didx_dummy, didx.at[c], isemd).wait()
                _transform(c, q)
                pltpu.async_copy(h_sh.at[sidx.at[c]], rows[c], gsem)

            def _step(i, carry):
                for u in range(IDEPTH):
                    j = i * IDEPTH + u
                    buf = rows[u % RDEPTH]
                    pltpu.make_async_copy(rows_dummy, buf, gsem).wait()
                    pltpu.async_copy(buf, acc_sh.at[didx.at[u]], ssem,
                                     add=True)

                    @pl.when(j + IDEPTH < n_chunks)
                    def _():
                        pltpu.async_copy(
                            src_hbm.at[wid, j + IDEPTH], sidx.at[u], isems)
                        pltpu.async_copy(
                            dst_hbm.at[wid, j + IDEPTH], didx.at[u], isemd)

                    @pl.when(j + RDEPTH < n_chunks)
                    def _():
                        pltpu.make_async_copy(
                            sidx_dummy, sidx.at[u], isems).wait()
                        pltpu.make_async_copy(
                            didx_dummy, didx.at[u], isemd).wait()
                        _transform((u + RDEPTH) % IDEPTH, q)
                        pltpu.make_async_copy(rows_dummy, buf, ssem).wait()
                        pltpu.async_copy(
                            h_sh.at[sidx.at[(u + RDEPTH) % IDEPTH]],
                            buf, gsem)
                return carry
            lax.fori_loop(0, n_chunks // IDEPTH, _step, 0)
            for _ in range(RDEPTH):
                pltpu.make_async_copy(rows_dummy, rows0, ssem).wait()

        plsc.subcore_barrier()
        pltpu.sync_copy(acc_sh.at[pl.ds(zbase, per_tile)],
                        acc_hbm.at[cid, pl.ds(zbase, per_tile)])

    return sc_scatter


def kernel(x, edge_index, W, b):
    n_src, d_in = x.shape
    n_out = n_src
    e = edge_index.shape[1]

    per_w = -(-e // NW)
    n_chunks = -(-(-(-per_w // CHUNK)) // IDEPTH) * IDEPTH
    per_w = n_chunks * CHUNK
    e_pad = per_w * NW

    n_acc = -(-(n_out + 1 + 64) // 128) * 128
    src = edge_index[0]
    dst = edge_index[1]
    pad = e_pad - e
    src_p = jnp.concatenate([src, jnp.zeros((pad,), jnp.int32)])
    dst_p = jnp.concatenate([dst, jnp.full((pad,), n_out, jnp.int32)])
    src3 = src_p.reshape(NW, n_chunks, CHUNK)
    dst3 = dst_p.reshape(NW, n_chunks, CHUNK)
    dst2 = dst_p.reshape(NW, per_w)

    blk = 2000
    h = pl.pallas_call(
        _matmul_body,
        grid=(n_src // blk,),
        in_specs=[
            pl.BlockSpec((blk, d_in), lambda i: (i, 0)),
            pl.BlockSpec((d_in, D), lambda i: (0, 0)),
        ],
        out_specs=pl.BlockSpec((blk, D), lambda i: (i, 0)),
        out_shape=jax.ShapeDtypeStruct((n_src, D), jnp.float32),
    )(x, W)
    h_pad = jnp.concatenate(
        [h, jnp.zeros((NPASS * HROWS - n_src, D), jnp.float32)])

    n_deg = -(-(n_out + 1) // 16) * 16
    cnt = _make_deg(per_w // DEGCH, n_deg)(dst2)
    acc = _make_sc_scatter(n_chunks, n_acc)(src3, dst3, h_pad)
    cnt_t = cnt.T

    fblk = 2000
    out = pl.pallas_call(
        _finalize_body,
        grid=(n_out // fblk,),
        in_specs=[
            pl.BlockSpec((NC, fblk, D), lambda i: (0, i, 0)),
            pl.BlockSpec((fblk, NW), lambda i: (i, 0)),
            pl.BlockSpec((1, D), lambda i: (0, 0)),
        ],
        out_specs=pl.BlockSpec((fblk, D), lambda i: (i, 0)),
        out_shape=jax.ShapeDtypeStruct((n_out, D), jnp.float32),
    )(acc, cnt_t, b.reshape(1, D))
    return out


# speedup vs baseline: 2.2487x; 2.0133x over previous
import functools

import jax
import jax.numpy as jnp
from jax import lax
from jax.experimental import pallas as pl
from jax.experimental.pallas import tpu as pltpu
from jax.experimental.pallas import tpu_sc as plsc

D = 128
NC = 2
NS = 16
NW = NC * NS
CHUNK = 64
RDEPTH = 3
IDEPTH = 8
HROWS = 2688
NPASS = 4
DEGCH = 1024


def _matmul_body(x_ref, w_ref, o_ref):
    o_ref[...] = jnp.dot(x_ref[...], w_ref[...],
                         preferred_element_type=jnp.float32)


def _finalize_body(acc_ref, cnt_ref, b_ref, o_ref):
    a = acc_ref[0, :, :] + acc_ref[1, :, :]
    cnt = jnp.sum(cnt_ref[...], axis=1, keepdims=True)
    scale = lax.rsqrt(cnt + 1.0)
    o_ref[...] = jnp.maximum(a * scale + b_ref[...], 0.0)


def _make_deg(n_flat, n_deg):
    mesh = plsc.VectorSubcoreMesh(core_axis_name="c", subcore_axis_name="s")

    @functools.partial(
        pl.kernel,
        mesh=mesh,
        compiler_params=pltpu.CompilerParams(needs_layout_passes=False),
        out_type=jax.ShapeDtypeStruct((NW, n_deg), jnp.float32),
        scratch_types=[
            pltpu.VMEM((DEGCH,), jnp.int32),
            pltpu.VMEM((DEGCH,), jnp.int32),
            pltpu.VMEM((n_deg,), jnp.float32),
            pltpu.SemaphoreType.DMA,
        ],
    )
    def deg_kernel(dst_hbm, cnt_hbm, dbuf0, dbuf1, deg_v, dsem):
        cid = lax.axis_index("c")
        sid = lax.axis_index("s")
        wid = cid * NS + sid

        def _zero_deg(i, carry):
            deg_v[pl.ds(i * 16, 16)] = jnp.zeros((16,), jnp.float32)
            return carry
        lax.fori_loop(0, n_deg // 16, _zero_deg, 0)

        pltpu.async_copy(dst_hbm.at[wid, pl.ds(0, DEGCH)], dbuf0, dsem)
        pltpu.async_copy(dst_hbm.at[wid, pl.ds(DEGCH, DEGCH)], dbuf1, dsem)
        bufs = (dbuf0, dbuf1)
        dummy = dst_hbm.at[0, pl.ds(0, DEGCH)]
        ones16 = jnp.ones((16,), jnp.float32)

        def _pair(i, carry):
            for u in range(2):
                k = i * 2 + u
                buf = bufs[u]
                pltpu.make_async_copy(dummy, buf, dsem).wait()

                def _hist(g, carry2):
                    dv = buf[pl.ds(g * 16, 16)]
                    plsc.addupdate_scatter(deg_v, [dv], ones16)
                    return carry2
                lax.fori_loop(0, DEGCH // 16, _hist, 0)

                @pl.when(k + 2 < n_flat)
                def _():
                    pltpu.async_copy(
                        dst_hbm.at[wid, pl.ds((k + 2) * DEGCH, DEGCH)],
                        buf, dsem)
            return carry
        lax.fori_loop(0, n_flat // 2, _pair, 0)
        pltpu.sync_copy(deg_v, cnt_hbm.at[wid])

    return deg_kernel


def _make_sc_scatter(n_chunks, n_acc):
    mesh = plsc.VectorSubcoreMesh(core_axis_name="c", subcore_axis_name="s")
    per_tile = n_acc // NS
    hrows_pt = HROWS // NS
    trash = n_acc - 64

    @functools.partial(
        pl.kernel,
        mesh=mesh,
        compiler_params=pltpu.CompilerParams(needs_layout_passes=False),
        out_type=jax.ShapeDtypeStruct((NC, n_acc, D), jnp.float32),
        scratch_types=[
            pltpu.VMEM((IDEPTH, CHUNK), jnp.int32),
            pltpu.VMEM((IDEPTH, CHUNK), jnp.int32),
            pltpu.VMEM((CHUNK, D), jnp.float32),
            pltpu.VMEM((CHUNK, D), jnp.float32),
            pltpu.VMEM((CHUNK, D), jnp.float32),
            pltpu.VMEM((2 * CHUNK + 16,), jnp.int32),
            pltpu.VMEM((2 * CHUNK + 16,), jnp.int32),
            pltpu.VMEM((CHUNK,), jnp.int32),
            pltpu.VMEM((CHUNK,), jnp.int32),
            pltpu.VMEM((CHUNK,), jnp.int32),
            pltpu.VMEM((CHUNK,), jnp.int32),
            pltpu.VMEM((CHUNK,), jnp.int32),
            pltpu.VMEM((CHUNK,), jnp.int32),
            pltpu.VMEM_SHARED((HROWS, D), jnp.float32),
            pltpu.VMEM_SHARED((n_acc, D), jnp.float32),
            pltpu.SemaphoreType.DMA,
            pltpu.SemaphoreType.DMA,
            pltpu.SemaphoreType.DMA,
            pltpu.SemaphoreType.DMA,
        ],
    )
    def sc_scatter(src_hbm, dst_hbm, h_hbm, acc_hbm,
                   sidx, didx, r0, r1, r2, qs, qd,
                   fs0, fs1, fs2, fd0, fd1, fd2, h_sh, acc_sh,
                   gsem, isems, isemd, ssem):
        cid = lax.axis_index("c")
        sid = lax.axis_index("s")
        wid = cid * NS + sid
        rows = (r0, r1, r2)
        fs = (fs0, fs1, fs2)
        fd = (fd0, fd1, fd2)
        rows0 = r0

        def _zero_row(i, carry):
            for v in range(D // 16):
                rows0[i, pl.ds(v * 16, 16)] = jnp.zeros((16,), jnp.float32)
            return carry
        lax.fori_loop(0, CHUNK, _zero_row, 0)

        zbase = sid * per_tile
        nfull, tail = divmod(per_tile, CHUNK)
        for k in range(nfull):
            pltpu.sync_copy(rows0, acc_sh.at[pl.ds(zbase + k * CHUNK, CHUNK)])
        if tail:
            pltpu.sync_copy(rows0.at[pl.ds(0, tail)],
                            acc_sh.at[pl.ds(zbase + nfull * CHUNK, tail)])

        rows_dummy = h_hbm.at[pl.ds(0, CHUNK)]
        sidx_dummy = src_hbm.at[0, 0]
        didx_dummy = dst_hbm.at[0, 0]
        iota16 = lax.iota(jnp.int32, 16)

        def _fire(b, f):
            bm1 = (b + RDEPTH - 1) % RDEPTH

            @pl.when(f >= 1)
            def _():
                pltpu.make_async_copy(rows_dummy, rows[bm1], gsem).wait()
                pltpu.async_copy(rows[bm1], acc_sh.at[fd[bm1]], ssem,
                                 add=True)

            @pl.when(f >= RDEPTH)
            def _():
                pltpu.make_async_copy(rows_dummy, rows[b], ssem).wait()

            for k in range(CHUNK // 16):
                sl = pl.ds(k * 16, 16)
                fs[b][sl] = qs[sl]
                fd[b][sl] = qd[sl]
            pltpu.async_copy(h_sh.at[fs[b]], rows[b], gsem)
            for k in range(CHUNK // 16):
                sv = qs[pl.ds(CHUNK + k * 16, 16)]
                dv = qd[pl.ds(CHUNK + k * 16, 16)]
                qs[pl.ds(k * 16, 16)] = sv
                qd[pl.ds(k * 16, 16)] = dv

        for q in range(NPASS):
            plsc.subcore_barrier()
            pltpu.sync_copy(
                h_hbm.at[pl.ds(q * HROWS + sid * hrows_pt, hrows_pt)],
                h_sh.at[pl.ds(sid * hrows_pt, hrows_pt)])
            plsc.subcore_barrier()

            for c in range(IDEPTH):
                pltpu.async_copy(src_hbm.at[wid, c], sidx.at[c], isems)
                pltpu.async_copy(dst_hbm.at[wid, c], didx.at[c], isemd)

            def _step(j, carry):
                pos, f = carry
                u = j & (IDEPTH - 1)
                pltpu.make_async_copy(sidx_dummy, sidx.at[u], isems).wait()
                pltpu.make_async_copy(didx_dummy, didx.at[u], isemd).wait()

                for v in range(CHUNK // 16):
                    sl = pl.ds(v * 16, 16)
                    sv = sidx[u, sl]
                    dv = didx[u, sl]
                    ls = sv - (q * HROWS)
                    ok = (ls >= 0) & (ls < HROWS)
                    plsc.store_compressed(qs.at[pl.ds(pos, 16)], ls, mask=ok)
                    plsc.store_compressed(qd.at[pl.ds(pos, 16)], dv, mask=ok)
                    pos = pos + jnp.sum(ok.astype(jnp.int32))

                @pl.when(j + IDEPTH < n_chunks)
                def _():
                    pltpu.async_copy(
                        src_hbm.at[wid, j + IDEPTH], sidx.at[u], isems)
                    pltpu.async_copy(
                        dst_hbm.at[wid, j + IDEPTH], didx.at[u], isemd)

                fired = pos >= CHUNK
                for b in range(RDEPTH):
                    @pl.when(fired & (lax.rem(f, RDEPTH) == b))
                    def _():
                        _fire(b, f)
                pos = jnp.where(fired, pos - CHUNK, pos)
                f = f + fired.astype(jnp.int32)
                return (pos, f)

            pos, f = lax.fori_loop(
                0, n_chunks, _step,
                (jnp.int32(0), jnp.int32(0)))

            for k in range(CHUNK // 16):
                qs[pl.ds(pos + k * 16, 16)] = jnp.zeros((16,), jnp.int32)
                qd[pl.ds(pos + k * 16, 16)] = trash + k * 16 + iota16
            for b in range(RDEPTH):
                @pl.when(lax.rem(f, RDEPTH) == b)
                def _():
                    _fire(b, f)
            f = f + 1

            pltpu.make_async_copy(rows_dummy, rows0, gsem).wait()
            for b in range(RDEPTH):
                @pl.when(lax.rem(f - 1, RDEPTH) == b)
                def _():
                    pltpu.async_copy(rows[b], acc_sh.at[fd[b]], ssem,
                                     add=True)
            for t in range(RDEPTH):
                @pl.when(f >= RDEPTH - t)
                def _():
                    pltpu.make_async_copy(rows_dummy, rows0, ssem).wait()

        plsc.subcore_barrier()
        pltpu.sync_copy(acc_sh.at[pl.ds(zbase, per_tile)],
                        acc_hbm.at[cid, pl.ds(zbase, per_tile)])

    return sc_scatter


def kernel(x, edge_index, W, b):
    n_src, d_in = x.shape
    n_out = n_src
    e = edge_index.shape[1]

    per_w = -(-e // NW)
    n_chunks = -(-(-(-per_w // CHUNK)) // IDEPTH) * IDEPTH
    per_w = n_chunks * CHUNK
    e_pad = per_w * NW

    n_acc = -(-(n_out + 1 + 64) // 128) * 128
    src = edge_index[0]
    dst = edge_index[1]
    pad = e_pad - e
    src_p = jnp.concatenate([src, jnp.zeros((pad,), jnp.int32)])
    dst_p = jnp.concatenate([dst, jnp.full((pad,), n_out, jnp.int32)])
    src3 = src_p.reshape(NW, n_chunks, CHUNK)
    dst3 = dst_p.reshape(NW, n_chunks, CHUNK)
    dst2 = dst_p.reshape(NW, per_w)

    blk = 2000
    h = pl.pallas_call(
        _matmul_body,
        grid=(n_src // blk,),
        in_specs=[
            pl.BlockSpec((blk, d_in), lambda i: (i, 0)),
            pl.BlockSpec((d_in, D), lambda i: (0, 0)),
        ],
        out_specs=pl.BlockSpec((blk, D), lambda i: (i, 0)),
        out_shape=jax.ShapeDtypeStruct((n_src, D), jnp.float32),
    )(x, W)
    h_pad = jnp.concatenate(
        [h, jnp.zeros((NPASS * HROWS - n_src, D), jnp.float32)])

    n_deg = -(-(n_out + 1) // 16) * 16
    cnt = _make_deg(per_w // DEGCH, n_deg)(dst2)
    acc = _make_sc_scatter(n_chunks, n_acc)(src3, dst3, h_pad)
    cnt_t = cnt.T

    fblk = 2000
    out = pl.pallas_call(
        _finalize_body,
        grid=(n_out // fblk,),
        in_specs=[
            pl.BlockSpec((NC, fblk, D), lambda i: (0, i, 0)),
            pl.BlockSpec((fblk, NW), lambda i: (i, 0)),
            pl.BlockSpec((1, D), lambda i: (0, 0)),
        ],
        out_specs=pl.BlockSpec((fblk, D), lambda i: (i, 0)),
        out_shape=jax.ShapeDtypeStruct((n_out, D), jnp.float32),
    )(acc, cnt_t, b.reshape(1, D))
    return out
